# baseline (speedup 1.0000x reference)
import numpy as np

import jax
import jax.numpy as jnp
from jax.experimental import pallas as pl
from jax.experimental.pallas import tpu as pltpu

_BF16 = jnp.bfloat16


def _round_up(x, m):
    return ((x + m - 1) // m) * m


def _fused_kernel(xq_ref, sd_ref, od_ref,
                  w15_ref, b1c_ref, wq0_ref, wq1_ref, b2_ref,
                  w3_ref, b3c_ref, w1s_ref, w1m_ref, w1o_ref, b1_ref,
                  w2f_ref, b2f_ref, w3f_ref, b3f_ref,
                  o_ref, xcat_ref, s1_ref, s2_ref):
    TB = sd_ref.shape[0]
    M1 = 8 * TB

    s1_ref[:, :, 0, :, :] = jnp.zeros_like(s1_ref[:, :, 0, :, :])
    s1_ref[:, :, 9, :, :] = jnp.zeros_like(s1_ref[:, :, 9, :, :])
    s1_ref[:, :, :, :, 0:128] = jnp.zeros_like(s1_ref[:, :, :, :, 0:128])
    s1_ref[:, :, :, :, 1152:1280] = jnp.zeros_like(
        s1_ref[:, :, :, :, 1152:1280])
    xcat_ref[...] = jnp.zeros_like(xcat_ref)
    for q in (0, 1):
        for s in (0, 1):
            for ki in range(5):
                a, p = ki // 2, ki % 2
                w = (a + s) % 2
                qi = 2 * w + p
                V0 = (a + s) // 2
                xs = xq_ref[V0:V0 + 8, qi, :, :]
                xcat_ref[:, ki * 128:ki * 128 + 108] = xs.reshape(M1, 108)
            acc = jnp.dot(xcat_ref[...], w15_ref[q],
                          preferred_element_type=jnp.float32)
            y = jnp.maximum(acc + b1c_ref[...], 0.0).astype(_BF16)
            s1_ref[s, q, 1:9, :, 128:1152] = y.reshape(8, TB, 1024)

    for oj2 in range(8):
        acc = jnp.zeros((M1, 128), jnp.float32)
        for ki2 in range(5):
            a2, p2 = ki2 // 2, ki2 % 2
            x0 = s1_ref[p2, 0, a2:a2 + 8, :, oj2 * 128:(oj2 + 3) * 128]
            acc = acc + jnp.dot(x0.reshape(M1, 384), wq0_ref[ki2],
                                preferred_element_type=jnp.float32)
            x1 = s1_ref[p2, 1, a2:a2 + 8, :, oj2 * 128:(oj2 + 2) * 128]
            acc = acc + jnp.dot(x1.reshape(M1, 256), wq1_ref[ki2],
                                preferred_element_type=jnp.float32)
        acc = acc + b2_ref[...]
        s2_ref[:, :, oj2 * 128:(oj2 + 1) * 128] = (
            acc.astype(_BF16).reshape(8, TB, 128))

    sp = jnp.zeros((TB, 64), jnp.float32)
    for oi2 in range(8):
        sp = sp + jnp.dot(s2_ref[oi2], w3_ref[oi2],
                          preferred_element_type=jnp.float32)
    sp = jnp.maximum(sp + b3c_ref[...], 0.0).astype(_BF16)
    h1 = (jnp.dot(sd_ref[...], w1s_ref[...], preferred_element_type=jnp.float32)
          + jnp.dot(sp, w1m_ref[...], preferred_element_type=jnp.float32)
          + jnp.dot(od_ref[...], w1o_ref[...], preferred_element_type=jnp.float32)
          + b1_ref[...])
    h2 = jnp.dot(jnp.maximum(h1, 0.0).astype(_BF16), w2f_ref[...],
                 preferred_element_type=jnp.float32) + b2f_ref[...]
    o_ref[...] = jnp.dot(jnp.maximum(h2, 0.0).astype(_BF16), w3f_ref[...],
                         preferred_element_type=jnp.float32) + b3f_ref[...]


def _band_conv1_weights(conv1_w):
    w1p = jnp.pad(conv1_w, ((0, 0), (0, 0), (0, 0), (0, 32)))
    outs = []
    for q in (0, 1):
        rows = []
        for ki in range(5):
            acc = jnp.zeros((3, 36, 8, 128), jnp.float32)
            for kj in range(5):
                o = 2 * q + kj
                P = np.zeros((8, 36), np.float32)
                P[np.arange(8), 4 * np.arange(8) + o] = 1.0
                acc = acc + jnp.einsum('pw,cn->cwpn', jnp.asarray(P),
                                       w1p[ki, kj])
            band = acc.reshape(108, 1024)
            rows.append(jnp.pad(band, ((0, 20), (0, 0))))
        outs.append(jnp.concatenate(rows, axis=0))
    return jnp.stack(outs).astype(_BF16)


def kernel(conv1_w, conv1_b, conv2_w, conv2_b, conv3_w, conv3_b,
           fc1_w, fc1_b, fc2_w, fc2_b, fc3_w, fc3_b,
           mask, subject_dist, object_dist):
    B = mask.shape[0]
    no = subject_dist.shape[1]

    xp = jnp.pad(mask.astype(_BF16), ((0, 0), (0, 0), (2, 2), (2, 2)))
    xq = jnp.transpose(xp, (2, 0, 1, 3)).reshape(36, B, 108)
    xq = xq.reshape(9, 4, B, 108)

    TB = 64
    bp = _round_up(B, TB)
    sd = subject_dist.astype(_BF16)
    od = object_dist.astype(_BF16)
    if bp > B:
        xq = jnp.pad(xq, ((0, 0), (0, 0), (0, bp - B), (0, 0)))
        sd = jnp.pad(sd, ((0, bp - B), (0, 0)))
        od = jnp.pad(od, ((0, bp - B), (0, 0)))

    w15 = _band_conv1_weights(conv1_w)
    b1c = jnp.tile(jnp.pad(conv1_b, (0, 32)), 8).reshape(1, 1024)
    w2p = jnp.pad(conv2_w.astype(jnp.float32),
                  ((0, 0), (0, 0), (0, 32), (0, 0)))
    wq0 = jnp.stack([jnp.concatenate([w2p[ki, 0], w2p[ki, 2], w2p[ki, 4]])
                     for ki in range(5)]).astype(_BF16)
    wq1 = jnp.stack([jnp.concatenate([w2p[ki, 1], w2p[ki, 3]])
                     for ki in range(5)]).astype(_BF16)
    w3r = conv3_w.reshape(8, 8 * 128, 64).astype(_BF16)
    w1 = fc1_w.astype(_BF16)
    w1s, w1m, w1o = w1[:no], w1[no:no + 64], w1[no + 64:no + 64 + no]

    flops = 2 * bp * (256 * 75 * 96 + 64 * 25 * 96 * 128 + 8192 * 64
                      + 96 * 256 + 256 * 256 + 512)
    cost = pl.CostEstimate(flops=flops, transcendentals=0,
                           bytes_accessed=int(xq.size * 2 + bp * 8 + 4e6))

    def res(shape):
        return pl.BlockSpec(shape, lambda i: (0,) * len(shape))


    out = pl.pallas_call(
        _fused_kernel,
        out_shape=jax.ShapeDtypeStruct((bp, 2), jnp.float32),
        grid_spec=pltpu.PrefetchScalarGridSpec(
            num_scalar_prefetch=0,
            grid=(bp // TB,),
            in_specs=[pl.BlockSpec((9, 4, TB, 108), lambda i: (0, 0, i, 0)),
                      pl.BlockSpec((TB, no), lambda i: (i, 0)),
                      pl.BlockSpec((TB, no), lambda i: (i, 0)),
                      res((2, 640, 1024)), res((1, 1024)),
                      res((5, 384, 128)), res((5, 256, 128)), res((1, 128)),
                      res((8, 1024, 64)), res((1, 64)),
                      res((no, 256)), res((64, 256)), res((no, 256)),
                      res((1, 256)),
                      res((256, 256)), res((1, 256)),
                      res((256, 2)), res((1, 2))],
            out_specs=pl.BlockSpec((TB, 2), lambda i: (i, 0)),
            scratch_shapes=[pltpu.VMEM((8 * TB, 640), _BF16),
                            pltpu.VMEM((2, 2, 10, TB, 1280), _BF16),
                            pltpu.VMEM((8, TB, 1024), _BF16)]),
        compiler_params=pltpu.CompilerParams(
            dimension_semantics=("parallel",)),
        cost_estimate=cost,
    )(xq, sd, od,
      w15, b1c, wq0, wq1, conv2_b.reshape(1, 128),
      w3r, conv3_b.reshape(1, 64),
      w1s, w1m, w1o, fc1_b.reshape(1, 256),
      fc2_w.astype(_BF16), fc2_b.reshape(1, 256),
      fc3_w.astype(_BF16), fc3_b.reshape(1, 2))
    return out[:B]


# speedup vs baseline: 1.1136x; 1.1136x over previous
import numpy as np

import jax
import jax.numpy as jnp
from jax.experimental import pallas as pl
from jax.experimental.pallas import tpu as pltpu

_BF16 = jnp.bfloat16


def _round_up(x, m):
    return ((x + m - 1) // m) * m


def _fused_kernel(xq_ref, sd_ref, od_ref,
                  w15_ref, b1c_ref, wp0_ref, wq1_ref, b2_ref,
                  w3_ref, b3c_ref, w1s_ref, w1m_ref, w1o_ref, b1_ref,
                  w2f_ref, b2f_ref, w3f_ref, b3f_ref,
                  o_ref, xcat_ref, s1_ref, s2_ref):
    TB = sd_ref.shape[0]
    M1 = 8 * TB

    s1_ref[:, :, 0, :, :] = jnp.zeros_like(s1_ref[:, :, 0, :, :])
    s1_ref[:, :, 9, :, :] = jnp.zeros_like(s1_ref[:, :, 9, :, :])
    s1_ref[:, :, :, :, 0:128] = jnp.zeros_like(s1_ref[:, :, :, :, 0:128])
    s1_ref[:, :, :, :, 1152:1280] = jnp.zeros_like(
        s1_ref[:, :, :, :, 1152:1280])
    xcat_ref[...] = jnp.zeros_like(xcat_ref)
    for q in (0, 1):
        for s in (0, 1):
            for ki in range(5):
                a, p = ki // 2, ki % 2
                w = (a + s) % 2
                qi = 2 * w + p
                V0 = (a + s) // 2
                xs = xq_ref[V0:V0 + 8, qi, :, :]
                xcat_ref[:, ki * 128:ki * 128 + 108] = xs.reshape(M1, 108)
            acc = jnp.dot(xcat_ref[...], w15_ref[q],
                          preferred_element_type=jnp.float32)
            y = jnp.maximum(acc + b1c_ref[...], 0.0).astype(_BF16)
            s1_ref[s, q, 1:9, :, 128:1152] = y.reshape(8, TB, 1024)

    for oj2 in range(0, 8, 2):
        acc = jnp.zeros((M1, 256), jnp.float32)
        for ki2 in range(5):
            a2, p2 = ki2 // 2, ki2 % 2
            x0 = s1_ref[p2, 0, a2:a2 + 8, :, oj2 * 128:(oj2 + 4) * 128]
            acc = acc + jnp.dot(x0.reshape(M1, 512), wp0_ref[ki2],
                                preferred_element_type=jnp.float32)
        a0 = jnp.zeros((M1, 128), jnp.float32)
        a1 = jnp.zeros((M1, 128), jnp.float32)
        for ki2 in range(5):
            a2, p2 = ki2 // 2, ki2 % 2
            x1 = s1_ref[p2, 1, a2:a2 + 8, :, oj2 * 128:(oj2 + 2) * 128]
            a0 = a0 + jnp.dot(x1.reshape(M1, 256), wq1_ref[ki2],
                              preferred_element_type=jnp.float32)
            x1 = s1_ref[p2, 1, a2:a2 + 8, :, (oj2 + 1) * 128:(oj2 + 3) * 128]
            a1 = a1 + jnp.dot(x1.reshape(M1, 256), wq1_ref[ki2],
                              preferred_element_type=jnp.float32)
        acc = acc + jnp.concatenate([a0, a1], axis=1) + b2_ref[...]
        s2_ref[:, :, oj2 * 128:(oj2 + 2) * 128] = (
            acc.astype(_BF16).reshape(8, TB, 256))

    sp = jnp.zeros((TB, 64), jnp.float32)
    for oi2 in range(8):
        sp = sp + jnp.dot(s2_ref[oi2], w3_ref[oi2],
                          preferred_element_type=jnp.float32)
    sp = jnp.maximum(sp + b3c_ref[...], 0.0).astype(_BF16)
    h1 = (jnp.dot(sd_ref[...], w1s_ref[...], preferred_element_type=jnp.float32)
          + jnp.dot(sp, w1m_ref[...], preferred_element_type=jnp.float32)
          + jnp.dot(od_ref[...], w1o_ref[...], preferred_element_type=jnp.float32)
          + b1_ref[...])
    h2 = jnp.dot(jnp.maximum(h1, 0.0).astype(_BF16), w2f_ref[...],
                 preferred_element_type=jnp.float32) + b2f_ref[...]
    o_ref[...] = jnp.dot(jnp.maximum(h2, 0.0).astype(_BF16), w3f_ref[...],
                         preferred_element_type=jnp.float32) + b3f_ref[...]


def _band_conv1_weights(conv1_w):
    w1p = jnp.pad(conv1_w, ((0, 0), (0, 0), (0, 0), (0, 32)))
    outs = []
    for q in (0, 1):
        rows = []
        for ki in range(5):
            acc = jnp.zeros((3, 36, 8, 128), jnp.float32)
            for kj in range(5):
                o = 2 * q + kj
                P = np.zeros((8, 36), np.float32)
                P[np.arange(8), 4 * np.arange(8) + o] = 1.0
                acc = acc + jnp.einsum('pw,cn->cwpn', jnp.asarray(P),
                                       w1p[ki, kj])
            band = acc.reshape(108, 1024)
            rows.append(jnp.pad(band, ((0, 20), (0, 0))))
        outs.append(jnp.concatenate(rows, axis=0))
    return jnp.stack(outs).astype(_BF16)


def kernel(conv1_w, conv1_b, conv2_w, conv2_b, conv3_w, conv3_b,
           fc1_w, fc1_b, fc2_w, fc2_b, fc3_w, fc3_b,
           mask, subject_dist, object_dist):
    B = mask.shape[0]
    no = subject_dist.shape[1]

    xp = jnp.pad(mask.astype(_BF16), ((0, 0), (0, 0), (2, 2), (2, 2)))
    xq = jnp.transpose(xp, (2, 0, 1, 3)).reshape(36, B, 108)
    xq = xq.reshape(9, 4, B, 108)

    TB = 64
    bp = _round_up(B, TB)
    sd = subject_dist.astype(_BF16)
    od = object_dist.astype(_BF16)
    if bp > B:
        xq = jnp.pad(xq, ((0, 0), (0, 0), (0, bp - B), (0, 0)))
        sd = jnp.pad(sd, ((0, bp - B), (0, 0)))
        od = jnp.pad(od, ((0, bp - B), (0, 0)))

    w15 = _band_conv1_weights(conv1_w)
    b1c = jnp.tile(jnp.pad(conv1_b, (0, 32)), 8).reshape(1, 1024)
    w2p = jnp.pad(conv2_w.astype(jnp.float32),
                  ((0, 0), (0, 0), (0, 32), (0, 0)))
    wq0 = jnp.stack([jnp.concatenate([w2p[ki, 0], w2p[ki, 2], w2p[ki, 4]])
                     for ki in range(5)])
    wp0 = jnp.concatenate(
        [jnp.pad(wq0, ((0, 0), (0, 128), (0, 0))),
         jnp.pad(wq0, ((0, 0), (128, 0), (0, 0)))],
        axis=2).astype(_BF16)
    wq1 = jnp.stack([jnp.concatenate([w2p[ki, 1], w2p[ki, 3]])
                     for ki in range(5)]).astype(_BF16)
    w3r = conv3_w.reshape(8, 8 * 128, 64).astype(_BF16)
    w1 = fc1_w.astype(_BF16)
    w1s, w1m, w1o = w1[:no], w1[no:no + 64], w1[no + 64:no + 64 + no]

    flops = 2 * bp * (256 * 75 * 96 + 64 * 25 * 96 * 128 + 8192 * 64
                      + 96 * 256 + 256 * 256 + 512)
    cost = pl.CostEstimate(flops=flops, transcendentals=0,
                           bytes_accessed=int(xq.size * 2 + bp * 8 + 4e6))

    def res(shape):
        return pl.BlockSpec(shape, lambda i: (0,) * len(shape))


    out = pl.pallas_call(
        _fused_kernel,
        out_shape=jax.ShapeDtypeStruct((bp, 2), jnp.float32),
        grid_spec=pltpu.PrefetchScalarGridSpec(
            num_scalar_prefetch=0,
            grid=(bp // TB,),
            in_specs=[pl.BlockSpec((9, 4, TB, 108), lambda i: (0, 0, i, 0)),
                      pl.BlockSpec((TB, no), lambda i: (i, 0)),
                      pl.BlockSpec((TB, no), lambda i: (i, 0)),
                      res((2, 640, 1024)), res((1, 1024)),
                      res((5, 512, 256)), res((5, 256, 128)), res((1, 256)),
                      res((8, 1024, 64)), res((1, 64)),
                      res((no, 256)), res((64, 256)), res((no, 256)),
                      res((1, 256)),
                      res((256, 256)), res((1, 256)),
                      res((256, 2)), res((1, 2))],
            out_specs=pl.BlockSpec((TB, 2), lambda i: (i, 0)),
            scratch_shapes=[pltpu.VMEM((8 * TB, 640), _BF16),
                            pltpu.VMEM((2, 2, 10, TB, 1280), _BF16),
                            pltpu.VMEM((8, TB, 1024), _BF16)]),
        compiler_params=pltpu.CompilerParams(
            dimension_semantics=("parallel",)),
        cost_estimate=cost,
    )(xq, sd, od,
      w15, b1c, wp0, wq1, jnp.tile(conv2_b, 2).reshape(1, 256),
      w3r, conv3_b.reshape(1, 64),
      w1s, w1m, w1o, fc1_b.reshape(1, 256),
      fc2_w.astype(_BF16), fc2_b.reshape(1, 256),
      fc3_w.astype(_BF16), fc3_b.reshape(1, 2))
    return out[:B]


# speedup vs baseline: 1.1374x; 1.0214x over previous
import numpy as np

import jax
import jax.numpy as jnp
from jax.experimental import pallas as pl
from jax.experimental.pallas import tpu as pltpu

_BF16 = jnp.bfloat16


def _round_up(x, m):
    return ((x + m - 1) // m) * m


def _fused_kernel(xq_ref, sd_ref, od_ref,
                  w15_ref, b1c_ref, wp0_ref, wq1_ref, b2_ref,
                  w3_ref, b3c_ref, w1s_ref, w1m_ref, w1o_ref, b1_ref,
                  w2f_ref, b2f_ref, w3f_ref, b3f_ref,
                  o_ref, xcat_ref, s1_ref, s2_ref):
    TB = sd_ref.shape[0]
    M1 = 8 * TB

    s1_ref[:, :, 0, :, :] = jnp.zeros_like(s1_ref[:, :, 0, :, :])
    s1_ref[:, :, 9, :, :] = jnp.zeros_like(s1_ref[:, :, 9, :, :])
    s1_ref[:, :, :, :, 0:128] = jnp.zeros_like(s1_ref[:, :, :, :, 0:128])
    s1_ref[:, :, :, :, 1152:1280] = jnp.zeros_like(
        s1_ref[:, :, :, :, 1152:1280])
    xcat_ref[...] = jnp.zeros_like(xcat_ref)
    for q in (0, 1):
        for s in (0, 1):
            for ki in range(5):
                a, p = ki // 2, ki % 2
                w = (a + s) % 2
                qi = 2 * w + p
                V0 = (a + s) // 2
                xs = xq_ref[V0:V0 + 8, qi, :, :]
                xcat_ref[:, ki * 128:ki * 128 + 108] = xs.reshape(M1, 108)
            acc = jnp.dot(xcat_ref[...], w15_ref[q],
                          preferred_element_type=jnp.float32)
            y = jnp.maximum(acc + b1c_ref[...], 0.0).astype(_BF16)
            s1_ref[s, q, 1:9, :, 128:1152] = y.reshape(8, TB, 1024)

    for oj2 in range(0, 8, 2):
        acc = jnp.zeros((M1, 256), jnp.float32)
        for ki2 in range(5):
            a2, p2 = ki2 // 2, ki2 % 2
            x0 = s1_ref[p2, 0, a2:a2 + 8, :, oj2 * 128:(oj2 + 4) * 128]
            acc = acc + jnp.dot(x0.reshape(M1, 512), wp0_ref[ki2],
                                preferred_element_type=jnp.float32)
        a0 = jnp.zeros((M1, 128), jnp.float32)
        a1 = jnp.zeros((M1, 128), jnp.float32)
        for ki2 in range(5):
            a2, p2 = ki2 // 2, ki2 % 2
            x1 = s1_ref[p2, 1, a2:a2 + 8, :, oj2 * 128:(oj2 + 2) * 128]
            a0 = a0 + jnp.dot(x1.reshape(M1, 256), wq1_ref[ki2],
                              preferred_element_type=jnp.float32)
            x1 = s1_ref[p2, 1, a2:a2 + 8, :, (oj2 + 1) * 128:(oj2 + 3) * 128]
            a1 = a1 + jnp.dot(x1.reshape(M1, 256), wq1_ref[ki2],
                              preferred_element_type=jnp.float32)
        acc = acc + jnp.concatenate([a0, a1], axis=1) + b2_ref[...]
        s2_ref[:, :, oj2 * 128:(oj2 + 2) * 128] = (
            acc.astype(_BF16).reshape(8, TB, 256))

    sp = jnp.zeros((TB, 64), jnp.float32)
    for oi2 in range(8):
        sp = sp + jnp.dot(s2_ref[oi2], w3_ref[oi2],
                          preferred_element_type=jnp.float32)
    sp = jnp.maximum(sp + b3c_ref[...], 0.0).astype(_BF16)
    h1 = (jnp.dot(sd_ref[...], w1s_ref[...], preferred_element_type=jnp.float32)
          + jnp.dot(sp, w1m_ref[...], preferred_element_type=jnp.float32)
          + jnp.dot(od_ref[...], w1o_ref[...], preferred_element_type=jnp.float32)
          + b1_ref[...])
    h2 = jnp.dot(jnp.maximum(h1, 0.0).astype(_BF16), w2f_ref[...],
                 preferred_element_type=jnp.float32) + b2f_ref[...]
    o_ref[...] = jnp.dot(jnp.maximum(h2, 0.0).astype(_BF16), w3f_ref[...],
                         preferred_element_type=jnp.float32) + b3f_ref[...]


def _band_conv1_weights(conv1_w):
    w1p = jnp.pad(conv1_w, ((0, 0), (0, 0), (0, 0), (0, 32)))
    outs = []
    for q in (0, 1):
        rows = []
        for ki in range(5):
            acc = jnp.zeros((3, 36, 8, 128), jnp.float32)
            for kj in range(5):
                o = 2 * q + kj
                P = np.zeros((8, 36), np.float32)
                P[np.arange(8), 4 * np.arange(8) + o] = 1.0
                acc = acc + jnp.einsum('pw,cn->cwpn', jnp.asarray(P),
                                       w1p[ki, kj])
            band = acc.reshape(108, 1024)
            rows.append(jnp.pad(band, ((0, 20), (0, 0))))
        outs.append(jnp.concatenate(rows, axis=0))
    return jnp.stack(outs).astype(_BF16)


def kernel(conv1_w, conv1_b, conv2_w, conv2_b, conv3_w, conv3_b,
           fc1_w, fc1_b, fc2_w, fc2_b, fc3_w, fc3_b,
           mask, subject_dist, object_dist):
    B = mask.shape[0]
    no = subject_dist.shape[1]

    xp = jnp.pad(mask.astype(_BF16), ((0, 0), (0, 0), (2, 2), (2, 2)))
    xq = jnp.transpose(xp, (2, 0, 1, 3)).reshape(36, B, 108)
    xq = xq.reshape(9, 4, B, 108)

    TB = 128
    bp = _round_up(B, TB)
    sd = subject_dist.astype(_BF16)
    od = object_dist.astype(_BF16)
    if bp > B:
        xq = jnp.pad(xq, ((0, 0), (0, 0), (0, bp - B), (0, 0)))
        sd = jnp.pad(sd, ((0, bp - B), (0, 0)))
        od = jnp.pad(od, ((0, bp - B), (0, 0)))

    w15 = _band_conv1_weights(conv1_w)
    b1c = jnp.tile(jnp.pad(conv1_b, (0, 32)), 8).reshape(1, 1024)
    w2p = jnp.pad(conv2_w.astype(jnp.float32),
                  ((0, 0), (0, 0), (0, 32), (0, 0)))
    wq0 = jnp.stack([jnp.concatenate([w2p[ki, 0], w2p[ki, 2], w2p[ki, 4]])
                     for ki in range(5)])
    wp0 = jnp.concatenate(
        [jnp.pad(wq0, ((0, 0), (0, 128), (0, 0))),
         jnp.pad(wq0, ((0, 0), (128, 0), (0, 0)))],
        axis=2).astype(_BF16)
    wq1 = jnp.stack([jnp.concatenate([w2p[ki, 1], w2p[ki, 3]])
                     for ki in range(5)]).astype(_BF16)
    w3r = conv3_w.reshape(8, 8 * 128, 64).astype(_BF16)
    w1 = fc1_w.astype(_BF16)
    w1s, w1m, w1o = w1[:no], w1[no:no + 64], w1[no + 64:no + 64 + no]

    flops = 2 * bp * (256 * 75 * 96 + 64 * 25 * 96 * 128 + 8192 * 64
                      + 96 * 256 + 256 * 256 + 512)
    cost = pl.CostEstimate(flops=flops, transcendentals=0,
                           bytes_accessed=int(xq.size * 2 + bp * 8 + 4e6))

    def res(shape):
        return pl.BlockSpec(shape, lambda i: (0,) * len(shape))


    out = pl.pallas_call(
        _fused_kernel,
        out_shape=jax.ShapeDtypeStruct((bp, 2), jnp.float32),
        grid_spec=pltpu.PrefetchScalarGridSpec(
            num_scalar_prefetch=0,
            grid=(bp // TB,),
            in_specs=[pl.BlockSpec((9, 4, TB, 108), lambda i: (0, 0, i, 0)),
                      pl.BlockSpec((TB, no), lambda i: (i, 0)),
                      pl.BlockSpec((TB, no), lambda i: (i, 0)),
                      res((2, 640, 1024)), res((1, 1024)),
                      res((5, 512, 256)), res((5, 256, 128)), res((1, 256)),
                      res((8, 1024, 64)), res((1, 64)),
                      res((no, 256)), res((64, 256)), res((no, 256)),
                      res((1, 256)),
                      res((256, 256)), res((1, 256)),
                      res((256, 2)), res((1, 2))],
            out_specs=pl.BlockSpec((TB, 2), lambda i: (i, 0)),
            scratch_shapes=[pltpu.VMEM((8 * TB, 640), _BF16),
                            pltpu.VMEM((2, 2, 10, TB, 1280), _BF16),
                            pltpu.VMEM((8, TB, 1024), _BF16)]),
        compiler_params=pltpu.CompilerParams(
            dimension_semantics=("parallel",)),
        cost_estimate=cost,
    )(xq, sd, od,
      w15, b1c, wp0, wq1, jnp.tile(conv2_b, 2).reshape(1, 256),
      w3r, conv3_b.reshape(1, 64),
      w1s, w1m, w1o, fc1_b.reshape(1, 256),
      fc2_w.astype(_BF16), fc2_b.reshape(1, 256),
      fc3_w.astype(_BF16), fc3_b.reshape(1, 2))
    return out[:B]


# speedup vs baseline: 1.1486x; 1.0098x over previous
import numpy as np

import jax
import jax.numpy as jnp
from jax.experimental import pallas as pl
from jax.experimental.pallas import tpu as pltpu

_BF16 = jnp.bfloat16


def _round_up(x, m):
    return ((x + m - 1) // m) * m


def _fused_kernel(xq_ref, sd_ref, od_ref,
                  w15_ref, b1c_ref, wp0_ref, wq1p_ref, b2_ref,
                  w3_ref, b3c_ref, w1s_ref, w1m_ref, w1o_ref, b1_ref,
                  w2f_ref, b2f_ref, w3f_ref, b3f_ref,
                  o_ref, xcat_ref, s1_ref, s2_ref):
    TB = sd_ref.shape[0]
    M1 = 8 * TB

    s1_ref[:, :, 0, :, :] = jnp.zeros_like(s1_ref[:, :, 0, :, :])
    s1_ref[:, :, 9, :, :] = jnp.zeros_like(s1_ref[:, :, 9, :, :])
    s1_ref[:, :, :, :, 0:128] = jnp.zeros_like(s1_ref[:, :, :, :, 0:128])
    s1_ref[:, :, :, :, 1152:1280] = jnp.zeros_like(
        s1_ref[:, :, :, :, 1152:1280])
    xcat_ref[...] = jnp.zeros_like(xcat_ref)
    for s in (0, 1):
        for ki in range(5):
            a, p = ki // 2, ki % 2
            w = (a + s) % 2
            qi = 2 * w + p
            V0 = (a + s) // 2
            xs = xq_ref[V0:V0 + 8, qi, :, :]
            xcat_ref[s * M1:s * M1 + M1, ki * 128:ki * 128 + 108] = (
                xs.reshape(M1, 108))
    for q in (0, 1):
        acc = jnp.dot(xcat_ref[...], w15_ref[q],
                      preferred_element_type=jnp.float32)
        y = jnp.maximum(acc + b1c_ref[...], 0.0).astype(_BF16)
        s1_ref[0, q, 1:9, :, 128:1152] = y[:M1].reshape(8, TB, 1024)
        s1_ref[1, q, 1:9, :, 128:1152] = y[M1:].reshape(8, TB, 1024)

    for oj2 in range(0, 8, 2):
        acc = jnp.zeros((M1, 256), jnp.float32)
        for ki2 in range(5):
            a2, p2 = ki2 // 2, ki2 % 2
            x0 = s1_ref[p2, 0, a2:a2 + 8, :, oj2 * 128:(oj2 + 4) * 128]
            acc = acc + jnp.dot(x0.reshape(M1, 512), wp0_ref[ki2],
                                preferred_element_type=jnp.float32)
        for ki2 in range(5):
            a2, p2 = ki2 // 2, ki2 % 2
            x1 = s1_ref[p2, 1, a2:a2 + 8, :, oj2 * 128:(oj2 + 3) * 128]
            acc = acc + jnp.dot(x1.reshape(M1, 384), wq1p_ref[ki2],
                                preferred_element_type=jnp.float32)
        acc = acc + b2_ref[...]
        s2_ref[:, :, oj2 * 128:(oj2 + 2) * 128] = (
            acc.astype(_BF16).reshape(8, TB, 256))

    sp = jnp.zeros((TB, 64), jnp.float32)
    for oi2 in range(8):
        sp = sp + jnp.dot(s2_ref[oi2], w3_ref[oi2],
                          preferred_element_type=jnp.float32)
    sp = jnp.maximum(sp + b3c_ref[...], 0.0).astype(_BF16)
    h1 = (jnp.dot(sd_ref[...], w1s_ref[...], preferred_element_type=jnp.float32)
          + jnp.dot(sp, w1m_ref[...], preferred_element_type=jnp.float32)
          + jnp.dot(od_ref[...], w1o_ref[...], preferred_element_type=jnp.float32)
          + b1_ref[...])
    h2 = jnp.dot(jnp.maximum(h1, 0.0).astype(_BF16), w2f_ref[...],
                 preferred_element_type=jnp.float32) + b2f_ref[...]
    o_ref[...] = jnp.dot(jnp.maximum(h2, 0.0).astype(_BF16), w3f_ref[...],
                         preferred_element_type=jnp.float32) + b3f_ref[...]


def _band_conv1_weights(conv1_w):
    w1p = jnp.pad(conv1_w, ((0, 0), (0, 0), (0, 0), (0, 32)))
    outs = []
    for q in (0, 1):
        rows = []
        for ki in range(5):
            acc = jnp.zeros((3, 36, 8, 128), jnp.float32)
            for kj in range(5):
                o = 2 * q + kj
                P = np.zeros((8, 36), np.float32)
                P[np.arange(8), 4 * np.arange(8) + o] = 1.0
                acc = acc + jnp.einsum('pw,cn->cwpn', jnp.asarray(P),
                                       w1p[ki, kj])
            band = acc.reshape(108, 1024)
            rows.append(jnp.pad(band, ((0, 20), (0, 0))))
        outs.append(jnp.concatenate(rows, axis=0))
    return jnp.stack(outs).astype(_BF16)


def kernel(conv1_w, conv1_b, conv2_w, conv2_b, conv3_w, conv3_b,
           fc1_w, fc1_b, fc2_w, fc2_b, fc3_w, fc3_b,
           mask, subject_dist, object_dist):
    B = mask.shape[0]
    no = subject_dist.shape[1]

    xp = jnp.pad(mask.astype(_BF16), ((0, 0), (0, 0), (2, 2), (2, 2)))
    xq = jnp.transpose(xp, (2, 0, 1, 3)).reshape(36, B, 108)
    xq = xq.reshape(9, 4, B, 108)

    TB = 128
    bp = _round_up(B, TB)
    sd = subject_dist.astype(_BF16)
    od = object_dist.astype(_BF16)
    if bp > B:
        xq = jnp.pad(xq, ((0, 0), (0, 0), (0, bp - B), (0, 0)))
        sd = jnp.pad(sd, ((0, bp - B), (0, 0)))
        od = jnp.pad(od, ((0, bp - B), (0, 0)))

    w15 = _band_conv1_weights(conv1_w)
    b1c = jnp.tile(jnp.pad(conv1_b, (0, 32)), 8).reshape(1, 1024)
    w2p = jnp.pad(conv2_w.astype(jnp.float32),
                  ((0, 0), (0, 0), (0, 32), (0, 0)))
    wq0 = jnp.stack([jnp.concatenate([w2p[ki, 0], w2p[ki, 2], w2p[ki, 4]])
                     for ki in range(5)])
    wp0 = jnp.concatenate(
        [jnp.pad(wq0, ((0, 0), (0, 128), (0, 0))),
         jnp.pad(wq0, ((0, 0), (128, 0), (0, 0)))],
        axis=2).astype(_BF16)
    wq1 = jnp.stack([jnp.concatenate([w2p[ki, 1], w2p[ki, 3]])
                     for ki in range(5)])
    wq1p = jnp.concatenate(
        [jnp.pad(wq1, ((0, 0), (0, 128), (0, 0))),
         jnp.pad(wq1, ((0, 0), (128, 0), (0, 0)))],
        axis=2).astype(_BF16)
    w3r = conv3_w.reshape(8, 8 * 128, 64).astype(_BF16)
    w1 = fc1_w.astype(_BF16)
    w1s, w1m, w1o = w1[:no], w1[no:no + 64], w1[no + 64:no + 64 + no]

    flops = 2 * bp * (256 * 75 * 96 + 64 * 25 * 96 * 128 + 8192 * 64
                      + 96 * 256 + 256 * 256 + 512)
    cost = pl.CostEstimate(flops=flops, transcendentals=0,
                           bytes_accessed=int(xq.size * 2 + bp * 8 + 4e6))

    def res(shape):
        return pl.BlockSpec(shape, lambda i: (0,) * len(shape))


    out = pl.pallas_call(
        _fused_kernel,
        out_shape=jax.ShapeDtypeStruct((bp, 2), jnp.float32),
        grid_spec=pltpu.PrefetchScalarGridSpec(
            num_scalar_prefetch=0,
            grid=(bp // TB,),
            in_specs=[pl.BlockSpec((9, 4, TB, 108), lambda i: (0, 0, i, 0)),
                      pl.BlockSpec((TB, no), lambda i: (i, 0)),
                      pl.BlockSpec((TB, no), lambda i: (i, 0)),
                      res((2, 640, 1024)), res((1, 1024)),
                      res((5, 512, 256)), res((5, 384, 256)), res((1, 256)),
                      res((8, 1024, 64)), res((1, 64)),
                      res((no, 256)), res((64, 256)), res((no, 256)),
                      res((1, 256)),
                      res((256, 256)), res((1, 256)),
                      res((256, 2)), res((1, 2))],
            out_specs=pl.BlockSpec((TB, 2), lambda i: (i, 0)),
            scratch_shapes=[pltpu.VMEM((16 * TB, 640), _BF16),
                            pltpu.VMEM((2, 2, 10, TB, 1280), _BF16),
                            pltpu.VMEM((8, TB, 1024), _BF16)]),
        compiler_params=pltpu.CompilerParams(
            dimension_semantics=("parallel",)),
        cost_estimate=cost,
    )(xq, sd, od,
      w15, b1c, wp0, wq1p, jnp.tile(conv2_b, 2).reshape(1, 256),
      w3r, conv3_b.reshape(1, 64),
      w1s, w1m, w1o, fc1_b.reshape(1, 256),
      fc2_w.astype(_BF16), fc2_b.reshape(1, 256),
      fc3_w.astype(_BF16), fc3_b.reshape(1, 2))
    return out[:B]
